# Initial kernel scaffold; baseline (speedup 1.0000x reference)
import jax
import jax.numpy as jnp
from jax.experimental import pallas as pl


def kernel(node_feat, edge_feat, edge_index, in_weight, rel_weight, loop_weight, loop_rel, bias):
    raise NotImplementedError("write your pallas kernel here")


# speedup vs baseline: 4.5040x; 4.5040x over previous
import functools

import jax
import jax.numpy as jnp
from jax import lax
from jax.experimental import pallas as pl
from jax.experimental.pallas import tpu as pltpu
from jax.experimental.pallas import tpu_sc as plsc

N_NODES = 10000
N_EDGES = 320000
D = 128

NC = 2
NS = 16
NW = NC * NS
C = 128
CHUNKS = N_EDGES // C
N_PAD = 10112
ROWS_PER_TILE = N_PAD // NS
ZROWS = 8
LANES = 16


def _sc_body(node_hbm, edge_hbm, src_hbm, dst_hbm, out_hbm,
             src_v, dst_v, ga_v, ef_v, zbuf, partial_sh, sem_g, sem_e):
    c = lax.axis_index("c")
    s = lax.axis_index("s")
    wid = c * NS + s

    def zrow(r, _):
        for u in range(D // LANES):
            zbuf[r, pl.ds(u * LANES, LANES)] = jnp.zeros((LANES,), jnp.float32)
        return 0
    lax.fori_loop(0, ZROWS, zrow, 0)

    def zcopy(k, _):
        pltpu.sync_copy(zbuf, partial_sh.at[pl.ds(s * ROWS_PER_TILE + k * ZROWS, ZROWS)])
        return 0
    lax.fori_loop(0, ROWS_PER_TILE // ZROWS, zcopy, 0)
    plsc.subcore_barrier()

    n_my = 78 + jnp.where(wid < CHUNKS % NW, 1, 0)

    def chunk_body(i, _):
        off = (wid + i * NW) * C
        pltpu.sync_copy(src_hbm.at[pl.ds(off, C)], src_v)
        pltpu.sync_copy(dst_hbm.at[pl.ds(off, C)], dst_v)
        g = pltpu.async_copy(node_hbm.at[src_v], ga_v, sem_g)
        e = pltpu.async_copy(edge_hbm.at[pl.ds(off, C)], ef_v, sem_e)
        g.wait()
        e.wait()

        def mul_row(r, _):
            for u in range(D // LANES):
                sl = pl.ds(u * LANES, LANES)
                ga_v[r, sl] = ga_v[r, sl] * ef_v[r, sl]
            return 0
        lax.fori_loop(0, C, mul_row, 0)
        pltpu.sync_copy(ga_v, partial_sh.at[dst_v], add=True)
        return 0

    lax.fori_loop(0, n_my, chunk_body, 0)
    plsc.subcore_barrier()

    base = s * ROWS_PER_TILE
    pltpu.sync_copy(partial_sh.at[pl.ds(base, ROWS_PER_TILE)],
                    out_hbm.at[pl.ds(c * N_PAD + base, ROWS_PER_TILE)])


_sc_segsum = functools.partial(
    pl.kernel,
    out_type=jax.ShapeDtypeStruct((NC * N_PAD, D), jnp.float32),
    mesh=plsc.VectorSubcoreMesh(core_axis_name="c", subcore_axis_name="s"),
    scratch_types=[
        pltpu.VMEM((C,), jnp.int32),
        pltpu.VMEM((C,), jnp.int32),
        pltpu.VMEM((C, D), jnp.float32),
        pltpu.VMEM((C, D), jnp.float32),
        pltpu.VMEM((ZROWS, D), jnp.float32),
        pltpu.VMEM_SHARED((N_PAD, D), jnp.float32),
        pltpu.SemaphoreType.DMA,
        pltpu.SemaphoreType.DMA,
    ],
)(_sc_body)


EBLK = 2560


def _edge_mm_body(e_ref, w_ref, o_ref):
    o_ref[...] = jnp.dot(e_ref[...], w_ref[...], preferred_element_type=jnp.float32)


NBLK = 2000


def _node_body(p_ref, nf_ref, wi_ref, wl_ref, lr_ref, b_ref, o_ref):
    agg = p_ref[0] + p_ref[1]
    aggw = jnp.dot(agg, wi_ref[...], preferred_element_type=jnp.float32)
    loop_msg = jnp.dot(nf_ref[...] * lr_ref[...], wl_ref[...],
                       preferred_element_type=jnp.float32)
    o_ref[...] = (aggw + loop_msg) * 0.3333333 + b_ref[...]


def kernel(node_feat, edge_feat, edge_index, in_weight, rel_weight, loop_weight,
           loop_rel, bias):
    src = edge_index[0].astype(jnp.int32)
    dst = edge_index[1].astype(jnp.int32)

    partials = _sc_segsum(node_feat, edge_feat, src, dst)

    out_edges = pl.pallas_call(
        _edge_mm_body,
        grid=(N_EDGES // EBLK,),
        in_specs=[pl.BlockSpec((EBLK, D), lambda i: (i, 0)),
                  pl.BlockSpec((D, D), lambda i: (0, 0))],
        out_specs=pl.BlockSpec((EBLK, D), lambda i: (i, 0)),
        out_shape=jax.ShapeDtypeStruct((N_EDGES, D), jnp.float32),
    )(edge_feat, rel_weight)

    p3 = partials.reshape(NC, N_PAD, D)
    out_nodes = pl.pallas_call(
        _node_body,
        grid=(N_NODES // NBLK,),
        in_specs=[pl.BlockSpec((NC, NBLK, D), lambda i: (0, i, 0)),
                  pl.BlockSpec((NBLK, D), lambda i: (i, 0)),
                  pl.BlockSpec((D, D), lambda i: (0, 0)),
                  pl.BlockSpec((D, D), lambda i: (0, 0)),
                  pl.BlockSpec((1, D), lambda i: (0, 0)),
                  pl.BlockSpec((1, D), lambda i: (0, 0))],
        out_specs=pl.BlockSpec((NBLK, D), lambda i: (i, 0)),
        out_shape=jax.ShapeDtypeStruct((N_NODES, D), jnp.float32),
    )(p3, node_feat, in_weight, loop_weight, loop_rel.reshape(1, D),
      bias.reshape(1, D))

    return (out_nodes, out_edges)


# speedup vs baseline: 6.2692x; 1.3919x over previous
import functools

import jax
import jax.numpy as jnp
from jax import lax
from jax.experimental import pallas as pl
from jax.experimental.pallas import tpu as pltpu
from jax.experimental.pallas import tpu_sc as plsc

N_NODES = 10000
N_EDGES = 320000
D = 128

NC = 2
NS = 16
NW = NC * NS
C = 64
CHUNKS = N_EDGES // C
N_PAD = 10112
CPT = 156
ROWS_PER_TILE = N_PAD // NS
ZROWS = 8
LANES = 16


def _sc_body(node_hbm, edge_hbm, src_hbm, dst_hbm, out_hbm,
             src_i, dst_i, ga2, ef2, zbuf, partial_sh,
             sem_i0, sem_i1, sem_g0, sem_g1, sem_e0, sem_e1):
    c = lax.axis_index("c")
    s = lax.axis_index("s")
    wid = c * NS + s
    sem_i = (sem_i0, sem_i1)
    sem_g = (sem_g0, sem_g1)
    sem_e = (sem_e0, sem_e1)

    def zrow(r, _):
        for u in range(D // LANES):
            zbuf[r, pl.ds(u * LANES, LANES)] = jnp.zeros((LANES,), jnp.float32)
        return 0
    lax.fori_loop(0, ZROWS, zrow, 0)

    def zcopy(k, _):
        pltpu.sync_copy(zbuf, partial_sh.at[pl.ds(s * ROWS_PER_TILE + k * ZROWS, ZROWS)])
        return 0
    lax.fori_loop(0, ROWS_PER_TILE // ZROWS, zcopy, 0)
    plsc.subcore_barrier()

    first = wid * CPT

    def fire_idx(i, b):
        off = (first + i) * C
        pltpu.async_copy(src_hbm.at[pl.ds(off, C)], src_i.at[b], sem_i[b])
        pltpu.async_copy(dst_hbm.at[pl.ds(off, C)], dst_i.at[b], sem_i[b])

    def wait_idx(b):
        pltpu.make_async_copy(src_hbm.at[pl.ds(0, C)], src_i.at[b], sem_i[b]).wait()
        pltpu.make_async_copy(dst_hbm.at[pl.ds(0, C)], dst_i.at[b], sem_i[b]).wait()

    def fire_ge(i, b):
        pltpu.async_copy(node_hbm.at[src_i.at[b]], ga2.at[b], sem_g[b])
        pltpu.async_copy(edge_hbm.at[pl.ds((first + i) * C, C)], ef2.at[b], sem_e[b])

    def wait_ge(b):
        pltpu.make_async_copy(node_hbm.at[src_i.at[b]], ga2.at[b], sem_g[b]).wait()
        pltpu.make_async_copy(edge_hbm.at[pl.ds(0, C)], ef2.at[b], sem_e[b]).wait()

    def mul_chunk(b):
        def mul_row(r, _):
            for u in range(D // LANES):
                sl = pl.ds(u * LANES, LANES)
                ga2[b, r, sl] = ga2[b, r, sl] * ef2[b, r, sl]
            return 0
        lax.fori_loop(0, C, mul_row, 0)

    fire_idx(0, 0)
    fire_idx(1, 1)
    wait_idx(0)
    fire_ge(0, 0)

    def gbody(g, _):
        for b in range(2):
            i = g * 2 + b
            ob = 1 - b

            @pl.when(i + 1 < CPT)
            def _():
                wait_idx(ob)
                fire_ge(i + 1, ob)

            wait_ge(b)
            mul_chunk(b)
            pltpu.sync_copy(ga2.at[b], partial_sh.at[dst_i.at[b]], add=True)

            @pl.when(i + 2 < CPT)
            def _():
                fire_idx(i + 2, b)
        return 0

    lax.fori_loop(0, CPT // 2, gbody, 0)

    @pl.when(wid < CHUNKS - NW * CPT)
    def _tail():
        ch = NW * CPT + wid
        fire_idx_off = ch * C
        pltpu.async_copy(src_hbm.at[pl.ds(fire_idx_off, C)], src_i.at[0], sem_i0)
        pltpu.async_copy(dst_hbm.at[pl.ds(fire_idx_off, C)], dst_i.at[0], sem_i0)
        wait_idx(0)
        g = pltpu.async_copy(node_hbm.at[src_i.at[0]], ga2.at[0], sem_g0)
        e = pltpu.async_copy(edge_hbm.at[pl.ds(ch * C, C)], ef2.at[0], sem_e0)
        g.wait()
        e.wait()
        mul_chunk(0)
        pltpu.sync_copy(ga2.at[0], partial_sh.at[dst_i.at[0]], add=True)

    plsc.subcore_barrier()

    base = s * ROWS_PER_TILE
    pltpu.sync_copy(partial_sh.at[pl.ds(base, ROWS_PER_TILE)],
                    out_hbm.at[pl.ds(c * N_PAD + base, ROWS_PER_TILE)])


_sc_segsum = functools.partial(
    pl.kernel,
    out_type=jax.ShapeDtypeStruct((NC * N_PAD, D), jnp.float32),
    mesh=plsc.VectorSubcoreMesh(core_axis_name="c", subcore_axis_name="s"),
    scratch_types=[
        pltpu.VMEM((2, C), jnp.int32),
        pltpu.VMEM((2, C), jnp.int32),
        pltpu.VMEM((2, C, D), jnp.float32),
        pltpu.VMEM((2, C, D), jnp.float32),
        pltpu.VMEM((ZROWS, D), jnp.float32),
        pltpu.VMEM_SHARED((N_PAD, D), jnp.float32),
        pltpu.SemaphoreType.DMA,
        pltpu.SemaphoreType.DMA,
        pltpu.SemaphoreType.DMA,
        pltpu.SemaphoreType.DMA,
        pltpu.SemaphoreType.DMA,
        pltpu.SemaphoreType.DMA,
    ],
)(_sc_body)


EBLK = 2560


def _edge_mm_body(e_ref, w_ref, o_ref):
    o_ref[...] = jnp.dot(e_ref[...], w_ref[...], preferred_element_type=jnp.float32)


NBLK = 2000


def _node_body(p_ref, nf_ref, wi_ref, wl_ref, lr_ref, b_ref, o_ref):
    agg = p_ref[0] + p_ref[1]
    aggw = jnp.dot(agg, wi_ref[...], preferred_element_type=jnp.float32)
    loop_msg = jnp.dot(nf_ref[...] * lr_ref[...], wl_ref[...],
                       preferred_element_type=jnp.float32)
    o_ref[...] = (aggw + loop_msg) * 0.3333333 + b_ref[...]


def kernel(node_feat, edge_feat, edge_index, in_weight, rel_weight, loop_weight,
           loop_rel, bias):
    src = edge_index[0].astype(jnp.int32)
    dst = edge_index[1].astype(jnp.int32)

    partials = _sc_segsum(node_feat, edge_feat, src, dst)

    out_edges = pl.pallas_call(
        _edge_mm_body,
        grid=(N_EDGES // EBLK,),
        in_specs=[pl.BlockSpec((EBLK, D), lambda i: (i, 0)),
                  pl.BlockSpec((D, D), lambda i: (0, 0))],
        out_specs=pl.BlockSpec((EBLK, D), lambda i: (i, 0)),
        out_shape=jax.ShapeDtypeStruct((N_EDGES, D), jnp.float32),
    )(edge_feat, rel_weight)

    p3 = partials.reshape(NC, N_PAD, D)
    out_nodes = pl.pallas_call(
        _node_body,
        grid=(N_NODES // NBLK,),
        in_specs=[pl.BlockSpec((NC, NBLK, D), lambda i: (0, i, 0)),
                  pl.BlockSpec((NBLK, D), lambda i: (i, 0)),
                  pl.BlockSpec((D, D), lambda i: (0, 0)),
                  pl.BlockSpec((D, D), lambda i: (0, 0)),
                  pl.BlockSpec((1, D), lambda i: (0, 0)),
                  pl.BlockSpec((1, D), lambda i: (0, 0))],
        out_specs=pl.BlockSpec((NBLK, D), lambda i: (i, 0)),
        out_shape=jax.ShapeDtypeStruct((N_NODES, D), jnp.float32),
    )(p3, node_feat, in_weight, loop_weight, loop_rel.reshape(1, D),
      bias.reshape(1, D))

    return (out_nodes, out_edges)


# speedup vs baseline: 6.6998x; 1.0687x over previous
import functools

import jax
import jax.numpy as jnp
from jax import lax
from jax.experimental import pallas as pl
from jax.experimental.pallas import tpu as pltpu
from jax.experimental.pallas import tpu_sc as plsc

N_NODES = 10000
N_EDGES = 320000
D = 128

NC = 2
NS = 16
NW = NC * NS
C = 64
CHUNKS = N_EDGES // C
N_PAD = 10112
CPT = 156
ROWS_PER_TILE = N_PAD // NS
ZROWS = 8
LANES = 16


def _sc_body(node_hbm, edge_hbm, src_hbm, dst_hbm, out_hbm,
             src_i, dst_i, sdst, ga2, ef2, zbuf, partial_sh,
             sem_i0, sem_i1, sem_g0, sem_g1, sem_e0, sem_e1, sem_s0, sem_s1):
    c = lax.axis_index("c")
    s = lax.axis_index("s")
    wid = c * NS + s
    sem_i = (sem_i0, sem_i1)
    sem_g = (sem_g0, sem_g1)
    sem_e = (sem_e0, sem_e1)
    sem_s = (sem_s0, sem_s1)

    def zrow(r, _):
        for u in range(D // LANES):
            zbuf[r, pl.ds(u * LANES, LANES)] = jnp.zeros((LANES,), jnp.float32)
        return 0
    lax.fori_loop(0, ZROWS, zrow, 0)

    def zcopy(k, _):
        pltpu.sync_copy(zbuf, partial_sh.at[pl.ds(s * ROWS_PER_TILE + k * ZROWS, ZROWS)])
        return 0
    lax.fori_loop(0, ROWS_PER_TILE // ZROWS, zcopy, 0)
    plsc.subcore_barrier()

    first = wid * CPT

    def fire_idx(i, b):
        off = (first + i) * C
        pltpu.async_copy(src_hbm.at[pl.ds(off, C)], src_i.at[b], sem_i[b])
        pltpu.async_copy(dst_hbm.at[pl.ds(off, C)], dst_i.at[b], sem_i[b])

    def wait_idx(b):
        pltpu.make_async_copy(src_hbm.at[pl.ds(0, C)], src_i.at[b], sem_i[b]).wait()
        pltpu.make_async_copy(dst_hbm.at[pl.ds(0, C)], dst_i.at[b], sem_i[b]).wait()

    def fire_ge(i, b):
        pltpu.async_copy(node_hbm.at[src_i.at[b]], ga2.at[b], sem_g[b])
        pltpu.async_copy(edge_hbm.at[pl.ds((first + i) * C, C)], ef2.at[b], sem_e[b])

    def wait_ge(b):
        pltpu.make_async_copy(node_hbm.at[src_i.at[b]], ga2.at[b], sem_g[b]).wait()
        pltpu.make_async_copy(edge_hbm.at[pl.ds(0, C)], ef2.at[b], sem_e[b]).wait()

    def wait_s(b):
        pltpu.make_async_copy(ga2.at[b], partial_sh.at[sdst.at[b]], sem_s[b]).wait()

    def mul_chunk(b):
        def mul_rows(r2, _):
            for v in range(2):
                r = r2 * 2 + v
                for u in range(D // LANES):
                    sl = pl.ds(u * LANES, LANES)
                    ga2[b, r, sl] = ga2[b, r, sl] * ef2[b, r, sl]
            return 0
        lax.fori_loop(0, C // 2, mul_rows, 0)

    def step(i, b, head=False):
        ob = 1 - b

        @pl.when(i + 1 < CPT)
        def _():
            wait_idx(ob)
            if not head:
                wait_s(ob)
            fire_ge(i + 1, ob)

        wait_ge(b)
        mul_chunk(b)
        for u in range(C // LANES):
            sl = pl.ds(u * LANES, LANES)
            sdst[b, sl] = dst_i[b, sl]
        pltpu.async_copy(ga2.at[b], partial_sh.at[sdst.at[b]], sem_s[b], add=True)

        @pl.when(i + 2 < CPT)
        def _():
            fire_idx(i + 2, b)

    fire_idx(0, 0)
    fire_idx(1, 1)
    wait_idx(0)
    fire_ge(0, 0)

    step(jnp.int32(0), 0, head=True)
    step(jnp.int32(1), 1)

    def gbody(g, _):
        for k in range(2):
            step(2 + g * 2 + k, k)
        return 0

    lax.fori_loop(0, (CPT - 2) // 2, gbody, 0)
    wait_s(0)
    wait_s(1)

    @pl.when(wid < CHUNKS - NW * CPT)
    def _tail():
        ch = NW * CPT + wid
        fire_idx_off = ch * C
        pltpu.async_copy(src_hbm.at[pl.ds(fire_idx_off, C)], src_i.at[0], sem_i0)
        pltpu.async_copy(dst_hbm.at[pl.ds(fire_idx_off, C)], dst_i.at[0], sem_i0)
        wait_idx(0)
        g = pltpu.async_copy(node_hbm.at[src_i.at[0]], ga2.at[0], sem_g0)
        e = pltpu.async_copy(edge_hbm.at[pl.ds(ch * C, C)], ef2.at[0], sem_e0)
        g.wait()
        e.wait()
        mul_chunk(0)
        pltpu.sync_copy(ga2.at[0], partial_sh.at[dst_i.at[0]], add=True)

    plsc.subcore_barrier()

    base = s * ROWS_PER_TILE
    pltpu.sync_copy(partial_sh.at[pl.ds(base, ROWS_PER_TILE)],
                    out_hbm.at[pl.ds(c * N_PAD + base, ROWS_PER_TILE)])


_sc_segsum = functools.partial(
    pl.kernel,
    out_type=jax.ShapeDtypeStruct((NC * N_PAD, D), jnp.float32),
    mesh=plsc.VectorSubcoreMesh(core_axis_name="c", subcore_axis_name="s"),
    scratch_types=[
        pltpu.VMEM((2, C), jnp.int32),
        pltpu.VMEM((2, C), jnp.int32),
        pltpu.VMEM((2, C), jnp.int32),
        pltpu.VMEM((2, C, D), jnp.float32),
        pltpu.VMEM((2, C, D), jnp.float32),
        pltpu.VMEM((ZROWS, D), jnp.float32),
        pltpu.VMEM_SHARED((N_PAD, D), jnp.float32),
        pltpu.SemaphoreType.DMA,
        pltpu.SemaphoreType.DMA,
        pltpu.SemaphoreType.DMA,
        pltpu.SemaphoreType.DMA,
        pltpu.SemaphoreType.DMA,
        pltpu.SemaphoreType.DMA,
        pltpu.SemaphoreType.DMA,
        pltpu.SemaphoreType.DMA,
    ],
)(_sc_body)


EBLK = 2560


def _edge_mm_body(e_ref, w_ref, o_ref):
    o_ref[...] = jnp.dot(e_ref[...], w_ref[...], preferred_element_type=jnp.float32)


NBLK = 2000


def _node_body(p_ref, nf_ref, wi_ref, wl_ref, lr_ref, b_ref, o_ref):
    agg = p_ref[0] + p_ref[1]
    aggw = jnp.dot(agg, wi_ref[...], preferred_element_type=jnp.float32)
    loop_msg = jnp.dot(nf_ref[...] * lr_ref[...], wl_ref[...],
                       preferred_element_type=jnp.float32)
    o_ref[...] = (aggw + loop_msg) * 0.3333333 + b_ref[...]


def kernel(node_feat, edge_feat, edge_index, in_weight, rel_weight, loop_weight,
           loop_rel, bias):
    src = edge_index[0].astype(jnp.int32)
    dst = edge_index[1].astype(jnp.int32)

    partials = _sc_segsum(node_feat, edge_feat, src, dst)

    out_edges = pl.pallas_call(
        _edge_mm_body,
        grid=(N_EDGES // EBLK,),
        in_specs=[pl.BlockSpec((EBLK, D), lambda i: (i, 0)),
                  pl.BlockSpec((D, D), lambda i: (0, 0))],
        out_specs=pl.BlockSpec((EBLK, D), lambda i: (i, 0)),
        out_shape=jax.ShapeDtypeStruct((N_EDGES, D), jnp.float32),
    )(edge_feat, rel_weight)

    p3 = partials.reshape(NC, N_PAD, D)
    out_nodes = pl.pallas_call(
        _node_body,
        grid=(N_NODES // NBLK,),
        in_specs=[pl.BlockSpec((NC, NBLK, D), lambda i: (0, i, 0)),
                  pl.BlockSpec((NBLK, D), lambda i: (i, 0)),
                  pl.BlockSpec((D, D), lambda i: (0, 0)),
                  pl.BlockSpec((D, D), lambda i: (0, 0)),
                  pl.BlockSpec((1, D), lambda i: (0, 0)),
                  pl.BlockSpec((1, D), lambda i: (0, 0))],
        out_specs=pl.BlockSpec((NBLK, D), lambda i: (i, 0)),
        out_shape=jax.ShapeDtypeStruct((N_NODES, D), jnp.float32),
    )(p3, node_feat, in_weight, loop_weight, loop_rel.reshape(1, D),
      bias.reshape(1, D))

    return (out_nodes, out_edges)


# speedup vs baseline: 7.0743x; 1.0559x over previous
import functools

import jax
import jax.numpy as jnp
from jax import lax
from jax.experimental import pallas as pl
from jax.experimental.pallas import tpu as pltpu
from jax.experimental.pallas import tpu_sc as plsc

N_NODES = 10000
N_EDGES = 320000
D = 128

NC = 2
NS = 16
NW = NC * NS
C = 64
CHUNKS = N_EDGES // C
N_PAD = 10112
CPT = 156
ROWS_PER_TILE = N_PAD // NS
ZROWS = 8
LANES = 16


def _sc_body(node_hbm, edge_hbm, src_hbm, dst_hbm, out_hbm,
             src_i, dst_i, ga3, ef2, zbuf, partial_sh,
             sem_i0, sem_i1, sem_i2, sem_i3, sem_g0, sem_g1, sem_g2,
             sem_e0, sem_e1, sem_s0, sem_s1, sem_s2):
    c = lax.axis_index("c")
    s = lax.axis_index("s")
    wid = c * NS + s
    sem_i = (sem_i0, sem_i1, sem_i2, sem_i3)
    sem_g = (sem_g0, sem_g1, sem_g2)
    sem_e = (sem_e0, sem_e1)
    sem_s = (sem_s0, sem_s1, sem_s2)

    def zrow(r, _):
        for u in range(D // LANES):
            zbuf[r, pl.ds(u * LANES, LANES)] = jnp.zeros((LANES,), jnp.float32)
        return 0
    lax.fori_loop(0, ZROWS, zrow, 0)

    def zcopy(k, _):
        pltpu.sync_copy(zbuf, partial_sh.at[pl.ds(s * ROWS_PER_TILE + k * ZROWS, ZROWS)])
        return 0
    lax.fori_loop(0, ROWS_PER_TILE // ZROWS, zcopy, 0)
    plsc.subcore_barrier()

    first = wid * CPT

    def fire_idx(i, sl):
        off = (first + i) * C
        pltpu.async_copy(src_hbm.at[pl.ds(off, C)], src_i.at[sl], sem_i[sl])
        pltpu.async_copy(dst_hbm.at[pl.ds(off, C)], dst_i.at[sl], sem_i[sl])

    def wait_idx(sl):
        pltpu.make_async_copy(src_hbm.at[pl.ds(0, C)], src_i.at[sl], sem_i[sl]).wait()
        pltpu.make_async_copy(dst_hbm.at[pl.ds(0, C)], dst_i.at[sl], sem_i[sl]).wait()

    def fire_g(isl, gsl):
        pltpu.async_copy(node_hbm.at[src_i.at[isl]], ga3.at[gsl], sem_g[gsl])

    def wait_g(gsl):
        pltpu.make_async_copy(node_hbm.at[src_i.at[0]], ga3.at[gsl], sem_g[gsl]).wait()

    def fire_e(i, esl):
        pltpu.async_copy(edge_hbm.at[pl.ds((first + i) * C, C)], ef2.at[esl], sem_e[esl])

    def wait_e(esl):
        pltpu.make_async_copy(edge_hbm.at[pl.ds(0, C)], ef2.at[esl], sem_e[esl]).wait()

    def fire_s(isl, gsl):
        pltpu.async_copy(ga3.at[gsl], partial_sh.at[dst_i.at[isl]], sem_s[gsl], add=True)

    def wait_s(gsl):
        pltpu.make_async_copy(ga3.at[gsl], partial_sh.at[dst_i.at[0]], sem_s[gsl]).wait()

    def mul_chunk(gsl, esl):
        def mul_rows(r2, _):
            for v in range(2):
                r = r2 * 2 + v
                for u in range(D // LANES):
                    sl = pl.ds(u * LANES, LANES)
                    ga3[gsl, r, sl] = ga3[gsl, r, sl] * ef2[esl, r, sl]
            return 0
        lax.fori_loop(0, C // 2, mul_rows, 0)

    fire_idx(0, 0)
    fire_idx(1, 1)
    fire_idx(2, 2)
    wait_idx(0)
    fire_g(0, 0)
    fire_e(0, 0)
    wait_idx(1)
    fire_g(1, 1)

    def step(i, k):
        isl, gsl, esl = k % 4, k % 3, k % 2
        isl2, gsl2 = (k + 2) % 4, (k + 2) % 3
        isl3 = (k + 3) % 4
        gsl_prev = (k + 2) % 3

        @pl.when(i + 2 < CPT)
        def _():
            wait_idx(isl2)

            @pl.when(i >= 1)
            def _():
                wait_s(gsl_prev)
            fire_g(isl2, gsl2)

        @pl.when(i + 1 < CPT)
        def _():
            fire_e(i + 1, (k + 1) % 2)

        @pl.when(i + 3 < CPT)
        def _():
            fire_idx(i + 3, isl3)

        wait_g(gsl)
        wait_e(esl)
        mul_chunk(gsl, esl)
        fire_s(isl, gsl)

    def gbody(g, _):
        for k in range(12):
            step(g * 12 + k, k)
        return 0

    lax.fori_loop(0, CPT // 12, gbody, 0)
    wait_s(0)
    wait_s(1)
    wait_s(2)

    @pl.when(wid < CHUNKS - NW * CPT)
    def _tail():
        ch = NW * CPT + wid
        off = ch * C
        pltpu.async_copy(src_hbm.at[pl.ds(off, C)], src_i.at[0], sem_i0)
        pltpu.async_copy(dst_hbm.at[pl.ds(off, C)], dst_i.at[0], sem_i0)
        wait_idx(0)
        fire_g(0, 0)
        fire_e(ch - first, 0)
        wait_g(0)
        wait_e(0)
        mul_chunk(0, 0)
        pltpu.sync_copy(ga3.at[0], partial_sh.at[dst_i.at[0]], add=True)

    plsc.subcore_barrier()

    base = s * ROWS_PER_TILE
    pltpu.sync_copy(partial_sh.at[pl.ds(base, ROWS_PER_TILE)],
                    out_hbm.at[pl.ds(c * N_PAD + base, ROWS_PER_TILE)])


_sc_segsum = functools.partial(
    pl.kernel,
    out_type=jax.ShapeDtypeStruct((NC * N_PAD, D), jnp.float32),
    mesh=plsc.VectorSubcoreMesh(core_axis_name="c", subcore_axis_name="s"),
    scratch_types=[
        pltpu.VMEM((4, C), jnp.int32),
        pltpu.VMEM((4, C), jnp.int32),
        pltpu.VMEM((3, C, D), jnp.float32),
        pltpu.VMEM((2, C, D), jnp.float32),
        pltpu.VMEM((ZROWS, D), jnp.float32),
        pltpu.VMEM_SHARED((N_PAD, D), jnp.float32),
        pltpu.SemaphoreType.DMA,
        pltpu.SemaphoreType.DMA,
        pltpu.SemaphoreType.DMA,
        pltpu.SemaphoreType.DMA,
        pltpu.SemaphoreType.DMA,
        pltpu.SemaphoreType.DMA,
        pltpu.SemaphoreType.DMA,
        pltpu.SemaphoreType.DMA,
        pltpu.SemaphoreType.DMA,
        pltpu.SemaphoreType.DMA,
        pltpu.SemaphoreType.DMA,
        pltpu.SemaphoreType.DMA,
    ],
)(_sc_body)


EBLK = 2560


def _edge_mm_body(e_ref, w_ref, o_ref):
    o_ref[...] = jnp.dot(e_ref[...], w_ref[...], preferred_element_type=jnp.float32)


NBLK = 2000


def _node_body(p_ref, nf_ref, wi_ref, wl_ref, lr_ref, b_ref, o_ref):
    agg = p_ref[0] + p_ref[1]
    aggw = jnp.dot(agg, wi_ref[...], preferred_element_type=jnp.float32)
    loop_msg = jnp.dot(nf_ref[...] * lr_ref[...], wl_ref[...],
                       preferred_element_type=jnp.float32)
    o_ref[...] = (aggw + loop_msg) * 0.3333333 + b_ref[...]


def kernel(node_feat, edge_feat, edge_index, in_weight, rel_weight, loop_weight,
           loop_rel, bias):
    src = edge_index[0].astype(jnp.int32)
    dst = edge_index[1].astype(jnp.int32)

    partials = _sc_segsum(node_feat, edge_feat, src, dst)

    out_edges = pl.pallas_call(
        _edge_mm_body,
        grid=(N_EDGES // EBLK,),
        in_specs=[pl.BlockSpec((EBLK, D), lambda i: (i, 0)),
                  pl.BlockSpec((D, D), lambda i: (0, 0))],
        out_specs=pl.BlockSpec((EBLK, D), lambda i: (i, 0)),
        out_shape=jax.ShapeDtypeStruct((N_EDGES, D), jnp.float32),
    )(edge_feat, rel_weight)

    p3 = partials.reshape(NC, N_PAD, D)
    out_nodes = pl.pallas_call(
        _node_body,
        grid=(N_NODES // NBLK,),
        in_specs=[pl.BlockSpec((NC, NBLK, D), lambda i: (0, i, 0)),
                  pl.BlockSpec((NBLK, D), lambda i: (i, 0)),
                  pl.BlockSpec((D, D), lambda i: (0, 0)),
                  pl.BlockSpec((D, D), lambda i: (0, 0)),
                  pl.BlockSpec((1, D), lambda i: (0, 0)),
                  pl.BlockSpec((1, D), lambda i: (0, 0))],
        out_specs=pl.BlockSpec((NBLK, D), lambda i: (i, 0)),
        out_shape=jax.ShapeDtypeStruct((N_NODES, D), jnp.float32),
    )(p3, node_feat, in_weight, loop_weight, loop_rel.reshape(1, D),
      bias.reshape(1, D))

    return (out_nodes, out_edges)
